# Initial kernel scaffold; baseline (speedup 1.0000x reference)
import functools

import jax
import jax.numpy as jnp
from jax import lax
from jax.experimental import pallas as pl
from jax.experimental.pallas import tpu as pltpu
from jax.experimental.pallas import tpu_sc as plsc

_D = 64
_T = 200
_B = 4096
_NC = 2
_NS = 16
_NW = _NC * _NS
_BPW = _B // _NW
_LANES = 16
_NV = _D // _LANES
_C0 = 128
_C1 = _T - _C0

_mesh = plsc.VectorSubcoreMesh(core_axis_name="c", subcore_axis_name="s")


@functools.partial(
    pl.kernel,
    out_type=jax.ShapeDtypeStruct((_B, _T, _D), jnp.float32),
    mesh=_mesh,
    scratch_types=[
        pltpu.VMEM((_BPW, _T), jnp.int32),
        pltpu.VMEM((_BPW, _D), jnp.float32),
        pltpu.VMEM((_T, _D), jnp.float32),
        pltpu.VMEM((_T, _D), jnp.float32),
        pltpu.SemaphoreType.DMA,
    ],
)
def _dummy_world_rnn(table_hbm, act_hbm, z0_hbm, out_hbm,
                     idx_v, z_v, rows_v, out_v, sem):
    wid = lax.axis_index("s") * _NC + lax.axis_index("c")
    base = wid * _BPW
    pltpu.sync_copy(act_hbm.at[pl.ds(base, _BPW)], idx_v)
    pltpu.sync_copy(z0_hbm.at[pl.ds(base, _BPW)], z_v)

    def elem(i, carry):
        cp0 = pltpu.async_copy(
            table_hbm.at[idx_v.at[i, pl.ds(0, _C0)]],
            rows_v.at[pl.ds(0, _C0)], sem)
        cp1 = pltpu.async_copy(
            table_hbm.at[idx_v.at[i, pl.ds(_C0, _C1)]],
            rows_v.at[pl.ds(_C0, _C1)], sem)
        cp0.wait()
        cp1.wait()

        accs = tuple(z_v[i, pl.ds(_LANES * j, _LANES)] for j in range(_NV))

        def step(t, acc):
            new = []
            for j in range(_NV):
                a = acc[j] + rows_v[t, pl.ds(_LANES * j, _LANES)] * 0.1
                out_v[t, pl.ds(_LANES * j, _LANES)] = a
                new.append(a)
            return tuple(new)

        lax.fori_loop(0, _T, step, accs)
        pltpu.sync_copy(out_v, out_hbm.at[base + i])
        return carry

    lax.fori_loop(0, _BPW, elem, 0)


def kernel(z0, action_seq, act_emb_weight):
    return _dummy_world_rnn(act_emb_weight, action_seq.astype(jnp.int32), z0)


# speedup vs baseline: 4.5486x; 4.5486x over previous
import functools

import jax
import jax.numpy as jnp
from jax import lax
from jax.experimental import pallas as pl
from jax.experimental.pallas import tpu as pltpu
from jax.experimental.pallas import tpu_sc as plsc

_D = 64
_T = 200
_B = 4096
_NC = 2
_NS = 16
_NW = _NC * _NS
_BPW = _B // _NW
_LANES = 16
_NV = _D // _LANES
_C0 = 128
_C1 = _T - _C0

_mesh = plsc.VectorSubcoreMesh(core_axis_name="c", subcore_axis_name="s")


@functools.partial(
    pl.kernel,
    out_type=jax.ShapeDtypeStruct((_B, _T, _D), jnp.float32),
    mesh=_mesh,
    compiler_params=pltpu.CompilerParams(use_tc_tiling_on_sc=False),
    scratch_types=[
        pltpu.VMEM((_BPW, _T), jnp.int32),
        pltpu.VMEM((_BPW, _D), jnp.float32),
        pltpu.VMEM((_T, _D), jnp.float32),
        pltpu.VMEM((_T, _D), jnp.float32),
        pltpu.SemaphoreType.DMA,
    ],
)
def _dummy_world_rnn(table_hbm, act_hbm, z0_hbm, out_hbm,
                     idx_v, z_v, rows_v, out_v, sem):
    wid = lax.axis_index("s") * _NC + lax.axis_index("c")
    base = wid * _BPW
    pltpu.sync_copy(act_hbm.at[pl.ds(base, _BPW)], idx_v)
    pltpu.sync_copy(z0_hbm.at[pl.ds(base, _BPW)], z_v)

    def elem(i, carry):
        cp0 = pltpu.async_copy(
            table_hbm.at[idx_v.at[i, pl.ds(0, _C0)]],
            rows_v.at[pl.ds(0, _C0)], sem)
        cp1 = pltpu.async_copy(
            table_hbm.at[idx_v.at[i, pl.ds(_C0, _C1)]],
            rows_v.at[pl.ds(_C0, _C1)], sem)
        cp0.wait()
        cp1.wait()

        accs = tuple(z_v[i, pl.ds(_LANES * j, _LANES)] for j in range(_NV))

        def step(t, acc):
            new = []
            for j in range(_NV):
                a = acc[j] + rows_v[t, pl.ds(_LANES * j, _LANES)] * 0.1
                out_v[t, pl.ds(_LANES * j, _LANES)] = a
                new.append(a)
            return tuple(new)

        lax.fori_loop(0, _T, step, accs)
        pltpu.sync_copy(out_v, out_hbm.at[base + i])
        return carry

    lax.fori_loop(0, _BPW, elem, 0)


def kernel(z0, action_seq, act_emb_weight):
    return _dummy_world_rnn(act_emb_weight, action_seq.astype(jnp.int32), z0)
